# Initial kernel scaffold; baseline (speedup 1.0000x reference)
import functools

import jax
import jax.numpy as jnp
from jax import lax
from jax.experimental import pallas as pl
from jax.experimental.pallas import tpu as pltpu
from jax.experimental.pallas import tpu_sc as plsc

_EMBED = 512
_NC = 2
_NS = 16
_NW = _NC * _NS
_C = 80
_NBUF = 2


@functools.cache
def _build(n_rows: int):
    per_w = n_rows // _NW
    n_chunks = per_w // _C
    n_groups = n_chunks // _NBUF
    assert per_w * _NW == n_rows and n_chunks * _C == per_w
    assert n_groups * _NBUF == n_chunks

    mesh = plsc.VectorSubcoreMesh(core_axis_name="c", subcore_axis_name="s")

    @functools.partial(
        pl.kernel,
        mesh=mesh,
        out_type=jax.ShapeDtypeStruct((n_rows, _EMBED), jnp.float32),
        scratch_types=[
            pltpu.VMEM((per_w,), jnp.int32),
            pltpu.VMEM((_NBUF, _C, _EMBED), jnp.float32),
            pltpu.SemaphoreType.DMA,
            pltpu.SemaphoreType.DMA,
        ],
    )
    def gather_kernel(idx_hbm, table_hbm, out_hbm, idx_v, rows_v, sem0, sem1):
        sems = [sem0, sem1]
        wid = lax.axis_index("s") * _NC + lax.axis_index("c")
        base = wid * per_w

        pltpu.sync_copy(idx_hbm.at[pl.ds(base, per_w)], idx_v)

        def start_gather(g, b):
            pltpu.async_copy(
                table_hbm.at[idx_v.at[pl.ds(g * _C, _C)]], rows_v.at[b], sems[b]
            )

        def wait_gather(b):
            pltpu.make_async_copy(
                table_hbm.at[idx_v.at[pl.ds(0, _C)]], rows_v.at[b], sems[b]
            ).wait()

        def start_write(g, b):
            pltpu.async_copy(
                rows_v.at[b], out_hbm.at[pl.ds(base + g * _C, _C)], sems[b]
            )

        def wait_write(g, b):
            pltpu.make_async_copy(
                rows_v.at[b], out_hbm.at[pl.ds(base + g * _C, _C)], sems[b]
            ).wait()

        for b in range(_NBUF):
            start_gather(b, b)

        def group(gi, carry):
            for b in range(_NBUF):
                g = gi * _NBUF + b
                wait_gather(b)
                start_write(g, b)
                wait_write(g, b)

                @pl.when(g + _NBUF < n_chunks)
                def _():
                    start_gather(g + _NBUF, b)

            return carry

        lax.fori_loop(0, n_groups, group, 0)

    return gather_kernel


def kernel(indices, table):
    b, l = indices.shape
    idx_flat = indices.reshape(-1).astype(jnp.int32)
    out = _build(idx_flat.shape[0])(idx_flat, table)
    return out.reshape(b, l, _EMBED)


# speedup vs baseline: 1.3040x; 1.3040x over previous
import functools

import jax
import jax.numpy as jnp
from jax import lax
from jax.experimental import pallas as pl
from jax.experimental.pallas import tpu as pltpu
from jax.experimental.pallas import tpu_sc as plsc

_EMBED = 512
_NC = 2
_NS = 16
_NW = _NC * _NS
_C = 80
_NBUF = 2


@functools.cache
def _build(n_rows: int):
    per_w = n_rows // _NW
    n_chunks = per_w // _C
    n_groups = n_chunks // _NBUF
    assert per_w * _NW == n_rows and n_chunks * _C == per_w
    assert n_groups * _NBUF == n_chunks

    mesh = plsc.VectorSubcoreMesh(
        core_axis_name="c", subcore_axis_name="s", num_cores=_NC, num_subcores=_NS
    )

    @functools.partial(
        pl.kernel,
        mesh=mesh,
        out_type=jax.ShapeDtypeStruct((n_rows, _EMBED), jnp.float32),
        scratch_types=[
            pltpu.VMEM((per_w,), jnp.int32),
            pltpu.VMEM((_NBUF, _C, _EMBED), jnp.float32),
            pltpu.SemaphoreType.DMA,
            pltpu.SemaphoreType.DMA,
        ],
    )
    def gather_kernel(idx_hbm, table_hbm, out_hbm, idx_v, rows_v, sem0, sem1):
        sems = [sem0, sem1]
        wid = lax.axis_index("s") * _NC + lax.axis_index("c")
        base = wid * per_w

        pltpu.sync_copy(idx_hbm.at[pl.ds(base, per_w)], idx_v)

        def start_gather(g, b):
            pltpu.async_copy(
                table_hbm.at[idx_v.at[pl.ds(g * _C, _C)]], rows_v.at[b], sems[b]
            )

        def wait_gather(b):
            pltpu.make_async_copy(
                table_hbm.at[idx_v.at[pl.ds(0, _C)]], rows_v.at[b], sems[b]
            ).wait()

        def start_write(g, b):
            pltpu.async_copy(
                rows_v.at[b], out_hbm.at[pl.ds(base + g * _C, _C)], sems[b]
            )

        def wait_write(g, b):
            pltpu.make_async_copy(
                rows_v.at[b], out_hbm.at[pl.ds(base + g * _C, _C)], sems[b]
            ).wait()

        for b in range(_NBUF):
            start_gather(b, b)

        def group(gi, carry):
            for b in range(_NBUF):
                g = gi * _NBUF + b
                wait_gather(b)
                start_write(g, b)
                wait_write(g, b)

                @pl.when(g + _NBUF < n_chunks)
                def _():
                    start_gather(g + _NBUF, b)

            return carry

        lax.fori_loop(0, n_groups, group, 0)

    return gather_kernel


def kernel(indices, table):
    b, l = indices.shape
    idx_flat = indices.reshape(-1).astype(jnp.int32)
    out = _build(idx_flat.shape[0])(idx_flat, table)
    return out.reshape(b, l, _EMBED)
